# baseline (speedup 1.0000x reference)
import jax
import jax.numpy as jnp
from jax import lax
from jax.experimental import pallas as pl
from jax.experimental.pallas import tpu as pltpu
from jax.experimental.pallas import tpu_sc as plsc

N = 10000
D = 128
EPS = 1e-3

NC = 2
NS = 16
EB = 128

N_PAD = 10112
ZROWS = N_PAD // NS
DEGW = 128

_mesh = plsc.VectorSubcoreMesh(core_axis_name="c", subcore_axis_name="s")


def _degree_body(dstp_hbm, out_hbm, dst_v, drow_v, ones_v, zb_v, acc):
    c = lax.axis_index("c")
    s = lax.axis_index("s")
    nb = dst_v.shape[0]
    pltpu.sync_copy(dstp_hbm.at[c, s], dst_v)

    e1 = jnp.full((16,), 1.0, jnp.float32)
    z16 = jnp.zeros((16,), jnp.float32)

    def fill(i, _):
        for k in range(DEGW // 16):
            ones_v[i, pl.ds(k * 16, 16)] = e1
        return _

    lax.fori_loop(0, EB, fill, None)

    def zfill(i, _):
        for k in range(DEGW // 16):
            zb_v[i, pl.ds(k * 16, 16)] = z16
        return _

    lax.fori_loop(0, EB, zfill, None)
    for t in range(ZROWS // EB):
        pltpu.sync_copy(zb_v, acc.at[pl.ds(s * ZROWS + t * EB, EB)])
    rem = ZROWS % EB
    if rem:
        pltpu.sync_copy(zb_v.at[pl.ds(0, rem)],
                        acc.at[pl.ds(s * ZROWS + (ZROWS // EB) * EB, rem)])
    plsc.subcore_barrier()

    def step(j, _):
        for k in range(EB // 16):
            drow_v[pl.ds(k * 16, 16)] = dst_v[j, pl.ds(k * 16, 16)]
        pltpu.sync_copy(ones_v, acc.at[drow_v], add=True)
        return _

    lax.fori_loop(0, nb, step, None)
    plsc.subcore_barrier()
    pltpu.sync_copy(acc.at[pl.ds(s * ZROWS, ZROWS)],
                    out_hbm.at[c, pl.ds(s * ZROWS, ZROWS)])


def _scatter_body(h_hbm, srcp_hbm, dstp_hbm, out_hbm,
                  dst_v, srow_v, drow_v, sidx0_v, sidx1_v,
                  buf0, buf1, sem0, sem1, ssem0, ssem1, acc):
    c = lax.axis_index("c")
    s = lax.axis_index("s")
    nb = dst_v.shape[0]
    pltpu.sync_copy(dstp_hbm.at[c, s], dst_v)

    z16 = jnp.zeros((16,), jnp.float32)

    def zfill(i, _):
        for k in range(D // 16):
            buf0[i, pl.ds(k * 16, 16)] = z16
        return _

    lax.fori_loop(0, EB, zfill, None)
    for t in range(ZROWS // EB):
        pltpu.sync_copy(buf0, acc.at[pl.ds(s * ZROWS + t * EB, EB)])
    rem = ZROWS % EB
    if rem:
        pltpu.sync_copy(buf0.at[pl.ds(0, rem)],
                        acc.at[pl.ds(s * ZROWS + (ZROWS // EB) * EB, rem)])
    plsc.subcore_barrier()

    def stage(dst_ref, arr_ref, j):
        for k in range(EB // 16):
            dst_ref[pl.ds(k * 16, 16)] = arr_ref[j, pl.ds(k * 16, 16)]

    def half(t, j, sidx, didx, buf, gsem, ssem):
        pltpu.sync_copy(srcp_hbm.at[c, s, j], sidx)

        @pl.when(t > 0)
        def _drain_prev():
            pltpu.make_async_copy(buf, acc.at[didx], ssem).wait()

        pltpu.async_copy(h_hbm.at[sidx], buf, gsem).wait()
        stage(didx, dst_v, j)
        pltpu.async_copy(buf, acc.at[didx], ssem, add=True)

    def pair(t, _):
        j0 = 2 * t
        half(t, j0, sidx0_v, srow_v, buf0, sem0, ssem0)
        half(t, j0 + 1, sidx1_v, drow_v, buf1, sem1, ssem1)
        return _

    lax.fori_loop(0, nb // 2, pair, None)
    pltpu.make_async_copy(buf0, acc.at[srow_v], ssem0).wait()
    pltpu.make_async_copy(buf1, acc.at[drow_v], ssem1).wait()
    plsc.subcore_barrier()
    pltpu.sync_copy(acc.at[pl.ds(s * ZROWS, ZROWS)],
                    out_hbm.at[c, pl.ds(s * ZROWS, ZROWS)])


def _make_sc_kernels(nb):
    deg = pl.kernel(
        _degree_body,
        out_type=jax.ShapeDtypeStruct((NC, N_PAD, DEGW), jnp.float32),
        mesh=_mesh,
        scratch_types=[
            pltpu.VMEM((nb, EB), jnp.int32),
            pltpu.VMEM((EB,), jnp.int32),
            pltpu.VMEM((EB, DEGW), jnp.float32),
            pltpu.VMEM((EB, DEGW), jnp.float32),
            pltpu.VMEM_SHARED((N_PAD, DEGW), jnp.float32),
        ],
    )
    scat = pl.kernel(
        _scatter_body,
        out_type=jax.ShapeDtypeStruct((NC, N_PAD, D), jnp.float32),
        mesh=_mesh,
        scratch_types=[
            pltpu.VMEM((nb, EB), jnp.int32),
            pltpu.VMEM((EB,), jnp.int32),
            pltpu.VMEM((EB,), jnp.int32),
            pltpu.VMEM((EB,), jnp.int32),
            pltpu.VMEM((EB,), jnp.int32),
            pltpu.VMEM((EB, D), jnp.float32),
            pltpu.VMEM((EB, D), jnp.float32),
            pltpu.SemaphoreType.DMA,
            pltpu.SemaphoreType.DMA,
            pltpu.SemaphoreType.DMA,
            pltpu.SemaphoreType.DMA,
            pltpu.VMEM_SHARED((N_PAD, D), jnp.float32),
        ],
    )
    return deg, scat


_BN_C = float(1.0 / (1.0 + EPS) ** 0.5)
_RB = 1000


def _premm_body(x_ref, d_ref, w_ref, o_ref):
    sc = lax.rsqrt(1.0 + d_ref[0, :, 0:1] + d_ref[1, :, 0:1])
    o_ref[...] = jnp.dot(x_ref[...] * sc, w_ref[...],
                         preferred_element_type=jnp.float32)


def _combmm_body(p_ref, hh_ref, d_ref, b_ref, g_ref, be_ref, w_ref, o_ref):
    sc = lax.rsqrt(1.0 + d_ref[0, :, 0:1] + d_ref[1, :, 0:1])
    agg = sc * (p_ref[0] + p_ref[1] + hh_ref[...])
    y = (agg + b_ref[...]) * (g_ref[...] * _BN_C) + be_ref[...]
    h = jnp.maximum(y, 0.0)
    o_ref[...] = jnp.dot(h * sc, w_ref[...],
                         preferred_element_type=jnp.float32)


def _final_body(p_ref, hh_ref, d_ref, b_ref, g_ref, be_ref, o_ref):
    sc = lax.rsqrt(1.0 + d_ref[0, :, 0:1] + d_ref[1, :, 0:1])
    agg = sc * (p_ref[0] + p_ref[1] + hh_ref[...])
    y = (agg + b_ref[...]) * (g_ref[...] * _BN_C) + be_ref[...]
    o_ref[...] = jnp.maximum(y, 0.0)


_row_spec = pl.BlockSpec((_RB, D), lambda i: (i, 0))
_deg_spec = pl.BlockSpec((NC, _RB, DEGW), lambda i: (0, i, 0))
_part_spec = pl.BlockSpec((NC, _RB, D), lambda i: (0, i, 0))
_w_spec = pl.BlockSpec((D, D), lambda i: (0, 0))
_vec_spec = pl.BlockSpec((1, D), lambda i: (0, 0))

_GRID = (N // _RB,)

_premm = pl.pallas_call(
    _premm_body,
    grid=_GRID,
    in_specs=[_row_spec, _deg_spec, _w_spec],
    out_specs=_row_spec,
    out_shape=jax.ShapeDtypeStruct((N, D), jnp.float32),
)

_combmm = pl.pallas_call(
    _combmm_body,
    grid=_GRID,
    in_specs=[_part_spec, _row_spec, _deg_spec,
              _vec_spec, _vec_spec, _vec_spec, _w_spec],
    out_specs=_row_spec,
    out_shape=jax.ShapeDtypeStruct((N, D), jnp.float32),
)

_final = pl.pallas_call(
    _final_body,
    grid=_GRID,
    in_specs=[_part_spec, _row_spec, _deg_spec,
              _vec_spec, _vec_spec, _vec_spec],
    out_specs=_row_spec,
    out_shape=jax.ShapeDtypeStruct((N, D), jnp.float32),
)


def kernel(x, edge_index, W1, b1, g1, be1, W2, b2, g2, be2, W3, b3, g3, be3):
    src = edge_index[0].astype(jnp.int32)
    dst = edge_index[1].astype(jnp.int32)
    e = src.shape[0]
    nb = -(-e // (NC * NS * EB))
    nb = -(-nb // 8) * 8
    e_pad = NC * NS * nb * EB
    pad = e_pad - e
    srcp = jnp.concatenate([src, jnp.zeros((pad,), jnp.int32)])
    dstp = jnp.concatenate([dst, jnp.full((pad,), N, jnp.int32)])
    srcp = srcp.reshape(NC, NS, nb, EB)
    dstp = dstp.reshape(NC, NS, nb, EB)

    sc_deg, sc_scat = _make_sc_kernels(nb)

    degp = sc_deg(dstp)
    b1r, g1r, be1r = b1.reshape(1, D), g1.reshape(1, D), be1.reshape(1, D)
    b2r, g2r, be2r = b2.reshape(1, D), g2.reshape(1, D), be2.reshape(1, D)
    b3r, g3r, be3r = b3.reshape(1, D), g3.reshape(1, D), be3.reshape(1, D)

    hh1 = _premm(x, degp, W1)
    p1 = sc_scat(hh1, srcp, dstp)
    hh2 = _combmm(p1, hh1, degp, b1r, g1r, be1r, W2)
    p2 = sc_scat(hh2, srcp, dstp)
    hh3 = _combmm(p2, hh2, degp, b2r, g2r, be2r, W3)
    p3 = sc_scat(hh3, srcp, dstp)
    return _final(p3, hh3, degp, b3r, g3r, be3r)


# speedup vs baseline: 1.0030x; 1.0030x over previous
import jax
import jax.numpy as jnp
from jax import lax
from jax.experimental import pallas as pl
from jax.experimental.pallas import tpu as pltpu
from jax.experimental.pallas import tpu_sc as plsc

N = 10000
D = 128
EPS = 1e-3

NC = 2
NS = 16
EB = 128

N_PAD = 10112
ZROWS = N_PAD // NS
DEGW = 128

_mesh = plsc.VectorSubcoreMesh(core_axis_name="c", subcore_axis_name="s")


def _degree_body(dstp_hbm, out_hbm, dst_v, drow_v, ones_v, zb_v, acc):
    c = lax.axis_index("c")
    s = lax.axis_index("s")
    nb = dst_v.shape[0]
    pltpu.sync_copy(dstp_hbm.at[c, s], dst_v)

    e1 = jnp.full((16,), 1.0, jnp.float32)
    z16 = jnp.zeros((16,), jnp.float32)

    def fill(i, _):
        for k in range(DEGW // 16):
            ones_v[i, pl.ds(k * 16, 16)] = e1
        return _

    lax.fori_loop(0, EB, fill, None)

    def zfill(i, _):
        for k in range(DEGW // 16):
            zb_v[i, pl.ds(k * 16, 16)] = z16
        return _

    lax.fori_loop(0, EB, zfill, None)
    for t in range(ZROWS // EB):
        pltpu.sync_copy(zb_v, acc.at[pl.ds(s * ZROWS + t * EB, EB)])
    rem = ZROWS % EB
    if rem:
        pltpu.sync_copy(zb_v.at[pl.ds(0, rem)],
                        acc.at[pl.ds(s * ZROWS + (ZROWS // EB) * EB, rem)])
    plsc.subcore_barrier()

    def step(j, _):
        for k in range(EB // 16):
            drow_v[pl.ds(k * 16, 16)] = dst_v[j, pl.ds(k * 16, 16)]
        pltpu.sync_copy(ones_v, acc.at[drow_v], add=True)
        return _

    lax.fori_loop(0, nb, step, None)
    plsc.subcore_barrier()
    pltpu.sync_copy(acc.at[pl.ds(s * ZROWS, ZROWS)],
                    out_hbm.at[c, pl.ds(s * ZROWS, ZROWS)])


def _scatter_body(h_hbm, srcp_hbm, dstp_hbm, out_hbm,
                  dst_v, srow_v, drow_v, sidx0_v, sidx1_v,
                  buf0, buf1, sem0, sem1, ssem0, ssem1, acc):
    c = lax.axis_index("c")
    s = lax.axis_index("s")
    nb = dst_v.shape[0]
    pltpu.sync_copy(dstp_hbm.at[c, s], dst_v)

    z16 = jnp.zeros((16,), jnp.float32)

    def zfill(i, _):
        for k in range(D // 16):
            buf0[i, pl.ds(k * 16, 16)] = z16
        return _

    lax.fori_loop(0, EB, zfill, None)
    for t in range(ZROWS // EB):
        pltpu.sync_copy(buf0, acc.at[pl.ds(s * ZROWS + t * EB, EB)])
    rem = ZROWS % EB
    if rem:
        pltpu.sync_copy(buf0.at[pl.ds(0, rem)],
                        acc.at[pl.ds(s * ZROWS + (ZROWS // EB) * EB, rem)])
    plsc.subcore_barrier()

    def stage(dst_ref, arr_ref, j):
        for k in range(EB // 16):
            dst_ref[pl.ds(k * 16, 16)] = arr_ref[j, pl.ds(k * 16, 16)]

    npairs = nb // 2
    pltpu.sync_copy(srcp_hbm.at[c, s, 0], sidx0_v)
    pltpu.sync_copy(srcp_hbm.at[c, s, 1], sidx1_v)

    def pair(t, _):
        j0 = 2 * t
        j1 = j0 + 1

        @pl.when(t > 0)
        def _drain_prev():
            pltpu.make_async_copy(srcp_hbm.at[c, s, j0], sidx0_v, sem1).wait()
            pltpu.make_async_copy(srcp_hbm.at[c, s, j1], sidx1_v, sem1).wait()
            pltpu.make_async_copy(buf0, acc.at[srow_v], ssem0).wait()
            pltpu.make_async_copy(buf1, acc.at[drow_v], ssem1).wait()

        pltpu.async_copy(h_hbm.at[sidx0_v], buf0, sem0)
        pltpu.async_copy(h_hbm.at[sidx1_v], buf1, sem0)
        pltpu.make_async_copy(h_hbm.at[sidx0_v], buf0, sem0).wait()
        pltpu.make_async_copy(h_hbm.at[sidx1_v], buf1, sem0).wait()

        @pl.when(t + 1 < npairs)
        def _prefetch_idx():
            pltpu.async_copy(srcp_hbm.at[c, s, j0 + 2], sidx0_v, sem1)
            pltpu.async_copy(srcp_hbm.at[c, s, j1 + 2], sidx1_v, sem1)

        stage(srow_v, dst_v, j0)
        pltpu.async_copy(buf0, acc.at[srow_v], ssem0, add=True)
        stage(drow_v, dst_v, j1)
        pltpu.async_copy(buf1, acc.at[drow_v], ssem1, add=True)
        return _

    lax.fori_loop(0, npairs, pair, None)
    pltpu.make_async_copy(buf0, acc.at[srow_v], ssem0).wait()
    pltpu.make_async_copy(buf1, acc.at[drow_v], ssem1).wait()
    plsc.subcore_barrier()
    pltpu.sync_copy(acc.at[pl.ds(s * ZROWS, ZROWS)],
                    out_hbm.at[c, pl.ds(s * ZROWS, ZROWS)])


def _make_sc_kernels(nb):
    deg = pl.kernel(
        _degree_body,
        out_type=jax.ShapeDtypeStruct((NC, N_PAD, DEGW), jnp.float32),
        mesh=_mesh,
        scratch_types=[
            pltpu.VMEM((nb, EB), jnp.int32),
            pltpu.VMEM((EB,), jnp.int32),
            pltpu.VMEM((EB, DEGW), jnp.float32),
            pltpu.VMEM((EB, DEGW), jnp.float32),
            pltpu.VMEM_SHARED((N_PAD, DEGW), jnp.float32),
        ],
    )
    scat = pl.kernel(
        _scatter_body,
        out_type=jax.ShapeDtypeStruct((NC, N_PAD, D), jnp.float32),
        mesh=_mesh,
        scratch_types=[
            pltpu.VMEM((nb, EB), jnp.int32),
            pltpu.VMEM((EB,), jnp.int32),
            pltpu.VMEM((EB,), jnp.int32),
            pltpu.VMEM((EB,), jnp.int32),
            pltpu.VMEM((EB,), jnp.int32),
            pltpu.VMEM((EB, D), jnp.float32),
            pltpu.VMEM((EB, D), jnp.float32),
            pltpu.SemaphoreType.DMA,
            pltpu.SemaphoreType.DMA,
            pltpu.SemaphoreType.DMA,
            pltpu.SemaphoreType.DMA,
            pltpu.VMEM_SHARED((N_PAD, D), jnp.float32),
        ],
    )
    return deg, scat


_BN_C = float(1.0 / (1.0 + EPS) ** 0.5)
_RB = 1000


def _premm_body(x_ref, d_ref, w_ref, o_ref):
    sc = lax.rsqrt(1.0 + d_ref[0, :, 0:1] + d_ref[1, :, 0:1])
    o_ref[...] = jnp.dot(x_ref[...] * sc, w_ref[...],
                         preferred_element_type=jnp.float32)


def _combmm_body(p_ref, hh_ref, d_ref, b_ref, g_ref, be_ref, w_ref, o_ref):
    sc = lax.rsqrt(1.0 + d_ref[0, :, 0:1] + d_ref[1, :, 0:1])
    agg = sc * (p_ref[0] + p_ref[1] + hh_ref[...])
    y = (agg + b_ref[...]) * (g_ref[...] * _BN_C) + be_ref[...]
    h = jnp.maximum(y, 0.0)
    o_ref[...] = jnp.dot(h * sc, w_ref[...],
                         preferred_element_type=jnp.float32)


def _final_body(p_ref, hh_ref, d_ref, b_ref, g_ref, be_ref, o_ref):
    sc = lax.rsqrt(1.0 + d_ref[0, :, 0:1] + d_ref[1, :, 0:1])
    agg = sc * (p_ref[0] + p_ref[1] + hh_ref[...])
    y = (agg + b_ref[...]) * (g_ref[...] * _BN_C) + be_ref[...]
    o_ref[...] = jnp.maximum(y, 0.0)


_row_spec = pl.BlockSpec((_RB, D), lambda i: (i, 0))
_deg_spec = pl.BlockSpec((NC, _RB, DEGW), lambda i: (0, i, 0))
_part_spec = pl.BlockSpec((NC, _RB, D), lambda i: (0, i, 0))
_w_spec = pl.BlockSpec((D, D), lambda i: (0, 0))
_vec_spec = pl.BlockSpec((1, D), lambda i: (0, 0))

_GRID = (N // _RB,)

_premm = pl.pallas_call(
    _premm_body,
    grid=_GRID,
    in_specs=[_row_spec, _deg_spec, _w_spec],
    out_specs=_row_spec,
    out_shape=jax.ShapeDtypeStruct((N, D), jnp.float32),
)

_combmm = pl.pallas_call(
    _combmm_body,
    grid=_GRID,
    in_specs=[_part_spec, _row_spec, _deg_spec,
              _vec_spec, _vec_spec, _vec_spec, _w_spec],
    out_specs=_row_spec,
    out_shape=jax.ShapeDtypeStruct((N, D), jnp.float32),
)

_final = pl.pallas_call(
    _final_body,
    grid=_GRID,
    in_specs=[_part_spec, _row_spec, _deg_spec,
              _vec_spec, _vec_spec, _vec_spec],
    out_specs=_row_spec,
    out_shape=jax.ShapeDtypeStruct((N, D), jnp.float32),
)


def kernel(x, edge_index, W1, b1, g1, be1, W2, b2, g2, be2, W3, b3, g3, be3):
    src = edge_index[0].astype(jnp.int32)
    dst = edge_index[1].astype(jnp.int32)
    e = src.shape[0]
    nb = -(-e // (NC * NS * EB))
    nb = -(-nb // 8) * 8
    e_pad = NC * NS * nb * EB
    pad = e_pad - e
    srcp = jnp.concatenate([src, jnp.zeros((pad,), jnp.int32)])
    dstp = jnp.concatenate([dst, jnp.full((pad,), N, jnp.int32)])
    srcp = srcp.reshape(NC, NS, nb, EB)
    dstp = dstp.reshape(NC, NS, nb, EB)

    sc_deg, sc_scat = _make_sc_kernels(nb)

    degp = sc_deg(dstp)
    b1r, g1r, be1r = b1.reshape(1, D), g1.reshape(1, D), be1.reshape(1, D)
    b2r, g2r, be2r = b2.reshape(1, D), g2.reshape(1, D), be2.reshape(1, D)
    b3r, g3r, be3r = b3.reshape(1, D), g3.reshape(1, D), be3.reshape(1, D)

    hh1 = _premm(x, degp, W1)
    p1 = sc_scat(hh1, srcp, dstp)
    hh2 = _combmm(p1, hh1, degp, b1r, g1r, be1r, W2)
    p2 = sc_scat(hh2, srcp, dstp)
    hh3 = _combmm(p2, hh2, degp, b2r, g2r, be2r, W3)
    p3 = sc_scat(hh3, srcp, dstp)
    return _final(p3, hh3, degp, b3r, g3r, be3r)


# speedup vs baseline: 1.2956x; 1.2917x over previous
import functools

import jax
import jax.numpy as jnp
from jax import lax
from jax.experimental import pallas as pl
from jax.experimental.pallas import tpu as pltpu
from jax.experimental.pallas import tpu_sc as plsc

N = 10000
D = 128
EPS = 1e-3

NC = 2
NS = 16
EB = 128

N_PAD = 10112
ZROWS = N_PAD // NS
DEGW = 128
_R0 = 1.0 / 6.0

_mesh = plsc.VectorSubcoreMesh(core_axis_name="c", subcore_axis_name="s")


def _degree_body(nb0, nb1, dstp_hbm, out_hbm, dst_v, drow_v, ones_v, acc):
    c = lax.axis_index("c")
    s = lax.axis_index("s")
    nb = jnp.where(c == 0, nb0, nb1)
    pltpu.sync_copy(dstp_hbm.at[c, s], dst_v)

    e1 = jnp.full((16,), 1.0, jnp.float32)
    z16 = jnp.zeros((16,), jnp.float32)

    def zfill(i, _):
        for k in range(DEGW // 16):
            ones_v[i, pl.ds(k * 16, 16)] = z16
        return _

    lax.fori_loop(0, EB, zfill, None)
    for t in range(ZROWS // EB):
        pltpu.sync_copy(ones_v, acc.at[pl.ds(s * ZROWS + t * EB, EB)])
    rem = ZROWS % EB
    if rem:
        pltpu.sync_copy(ones_v.at[pl.ds(0, rem)],
                        acc.at[pl.ds(s * ZROWS + (ZROWS // EB) * EB, rem)])
    plsc.subcore_barrier()

    def fill(i, _):
        for k in range(DEGW // 16):
            ones_v[i, pl.ds(k * 16, 16)] = e1
        return _

    lax.fori_loop(0, EB, fill, None)

    def step(j, _):
        for k in range(EB // 16):
            drow_v[pl.ds(k * 16, 16)] = dst_v[j, pl.ds(k * 16, 16)]
        pltpu.sync_copy(ones_v, acc.at[drow_v], add=True)
        return _

    lax.fori_loop(0, nb, step, None)
    plsc.subcore_barrier()
    pltpu.sync_copy(acc.at[pl.ds(s * ZROWS, ZROWS)],
                    out_hbm.at[c, pl.ds(s * ZROWS, ZROWS)])


def _scatter_body(nb0, nb1, h_hbm, srcp_hbm, dstp_hbm, out_hbm,
                  srow_v, drow_v, sidx0_v, sidx1_v, pidx0_v, pidx1_v,
                  buf0, buf1, sem0, sem1, ssem0, ssem1, acc):
    c = lax.axis_index("c")
    s = lax.axis_index("s")
    nb = jnp.where(c == 0, nb0, nb1)

    z16 = jnp.zeros((16,), jnp.float32)

    def zfill(i, _):
        for k in range(D // 16):
            buf0[i, pl.ds(k * 16, 16)] = z16
        return _

    lax.fori_loop(0, EB, zfill, None)
    for t in range(ZROWS // EB):
        pltpu.sync_copy(buf0, acc.at[pl.ds(s * ZROWS + t * EB, EB)])
    rem = ZROWS % EB
    if rem:
        pltpu.sync_copy(buf0.at[pl.ds(0, rem)],
                        acc.at[pl.ds(s * ZROWS + (ZROWS // EB) * EB, rem)])
    plsc.subcore_barrier()

    def copy_row(dst_ref, src_ref):
        for k in range(EB // 16):
            dst_ref[pl.ds(k * 16, 16)] = src_ref[pl.ds(k * 16, 16)]

    npairs = nb // 2
    pltpu.sync_copy(srcp_hbm.at[c, s, 0], sidx0_v)
    pltpu.sync_copy(srcp_hbm.at[c, s, 1], sidx1_v)
    pltpu.sync_copy(dstp_hbm.at[c, s, 0], pidx0_v)
    pltpu.sync_copy(dstp_hbm.at[c, s, 1], pidx1_v)

    def pair(t, _):
        j0 = 2 * t
        j1 = j0 + 1

        @pl.when(t > 0)
        def _drain_prev():
            pltpu.make_async_copy(srcp_hbm.at[c, s, j0], sidx0_v, sem1).wait()
            pltpu.make_async_copy(srcp_hbm.at[c, s, j1], sidx1_v, sem1).wait()
            pltpu.make_async_copy(dstp_hbm.at[c, s, j0], pidx0_v, sem1).wait()
            pltpu.make_async_copy(dstp_hbm.at[c, s, j1], pidx1_v, sem1).wait()
            pltpu.make_async_copy(buf0, acc.at[srow_v], ssem0).wait()
            pltpu.make_async_copy(buf1, acc.at[drow_v], ssem1).wait()

        pltpu.async_copy(h_hbm.at[sidx0_v], buf0, sem0)
        pltpu.async_copy(h_hbm.at[sidx1_v], buf1, sem0)
        pltpu.make_async_copy(h_hbm.at[sidx0_v], buf0, sem0).wait()
        pltpu.make_async_copy(h_hbm.at[sidx1_v], buf1, sem0).wait()

        copy_row(srow_v, pidx0_v)
        copy_row(drow_v, pidx1_v)

        @pl.when(t + 1 < npairs)
        def _prefetch_idx():
            pltpu.async_copy(srcp_hbm.at[c, s, j0 + 2], sidx0_v, sem1)
            pltpu.async_copy(srcp_hbm.at[c, s, j1 + 2], sidx1_v, sem1)
            pltpu.async_copy(dstp_hbm.at[c, s, j0 + 2], pidx0_v, sem1)
            pltpu.async_copy(dstp_hbm.at[c, s, j1 + 2], pidx1_v, sem1)

        pltpu.async_copy(buf0, acc.at[srow_v], ssem0, add=True)
        pltpu.async_copy(buf1, acc.at[drow_v], ssem1, add=True)
        return _

    lax.fori_loop(0, npairs, pair, None)
    pltpu.make_async_copy(buf0, acc.at[srow_v], ssem0).wait()
    pltpu.make_async_copy(buf1, acc.at[drow_v], ssem1).wait()
    plsc.subcore_barrier()
    pltpu.sync_copy(acc.at[pl.ds(s * ZROWS, ZROWS)],
                    out_hbm.at[c, pl.ds(s * ZROWS, ZROWS)])


def _make_sc_kernels(nb, nb0, nb1):
    deg = pl.kernel(
        functools.partial(_degree_body, nb0, nb1),
        out_type=jax.ShapeDtypeStruct((NC, N_PAD, DEGW), jnp.float32),
        mesh=_mesh,
        scratch_types=[
            pltpu.VMEM((nb, EB), jnp.int32),
            pltpu.VMEM((EB,), jnp.int32),
            pltpu.VMEM((EB, DEGW), jnp.float32),
            pltpu.VMEM_SHARED((N_PAD, DEGW), jnp.float32),
        ],
    )
    scat = pl.kernel(
        functools.partial(_scatter_body, nb0, nb1),
        out_type=jax.ShapeDtypeStruct((NC, N_PAD, D), jnp.float32),
        mesh=_mesh,
        scratch_types=[
            pltpu.VMEM((EB,), jnp.int32),
            pltpu.VMEM((EB,), jnp.int32),
            pltpu.VMEM((EB,), jnp.int32),
            pltpu.VMEM((EB,), jnp.int32),
            pltpu.VMEM((EB,), jnp.int32),
            pltpu.VMEM((EB,), jnp.int32),
            pltpu.VMEM((EB, D), jnp.float32),
            pltpu.VMEM((EB, D), jnp.float32),
            pltpu.SemaphoreType.DMA,
            pltpu.SemaphoreType.DMA,
            pltpu.SemaphoreType.DMA,
            pltpu.SemaphoreType.DMA,
            pltpu.VMEM_SHARED((N_PAD, D), jnp.float32),
        ],
    )
    return deg, scat


_BN_C = float(1.0 / (1.0 + EPS) ** 0.5)
_RB = 1000


def _premm_body(x_ref, d_ref, w_ref, o_ref):
    sc = lax.rsqrt(1.0 + d_ref[0, :, 0:1] + d_ref[1, :, 0:1])
    o_ref[...] = jnp.dot(x_ref[...] * sc, w_ref[...],
                         preferred_element_type=jnp.float32)


def _combmm_body(p_ref, hh_ref, d_ref, b_ref, g_ref, be_ref, w_ref, o_ref):
    sc = lax.rsqrt(1.0 + d_ref[0, :, 0:1] + d_ref[1, :, 0:1])
    agg = sc * (p_ref[0] + p_ref[1] + hh_ref[...])
    y = (agg + b_ref[...]) * (g_ref[...] * _BN_C) + be_ref[...]
    h = jnp.maximum(y, 0.0)
    o_ref[...] = jnp.dot(h * sc, w_ref[...],
                         preferred_element_type=jnp.float32)


def _final_body(p_ref, hh_ref, d_ref, b_ref, g_ref, be_ref, o_ref):
    sc = lax.rsqrt(1.0 + d_ref[0, :, 0:1] + d_ref[1, :, 0:1])
    agg = sc * (p_ref[0] + p_ref[1] + hh_ref[...])
    y = (agg + b_ref[...]) * (g_ref[...] * _BN_C) + be_ref[...]
    o_ref[...] = jnp.maximum(y, 0.0)


_row_spec = pl.BlockSpec((_RB, D), lambda i: (i, 0))
_deg_spec = pl.BlockSpec((NC, _RB, DEGW), lambda i: (0, i, 0))
_part_spec = pl.BlockSpec((NC, _RB, D), lambda i: (0, i, 0))
_w_spec = pl.BlockSpec((D, D), lambda i: (0, 0))
_vec_spec = pl.BlockSpec((1, D), lambda i: (0, 0))

_GRID = (N // _RB,)

_premm = pl.pallas_call(
    _premm_body,
    grid=_GRID,
    in_specs=[_row_spec, _deg_spec, _w_spec],
    out_specs=_row_spec,
    out_shape=jax.ShapeDtypeStruct((N, D), jnp.float32),
)

_combmm = pl.pallas_call(
    _combmm_body,
    grid=_GRID,
    in_specs=[_part_spec, _row_spec, _deg_spec,
              _vec_spec, _vec_spec, _vec_spec, _w_spec],
    out_specs=_row_spec,
    out_shape=jax.ShapeDtypeStruct((N, D), jnp.float32),
)

_final = pl.pallas_call(
    _final_body,
    grid=_GRID,
    in_specs=[_part_spec, _row_spec, _deg_spec,
              _vec_spec, _vec_spec, _vec_spec],
    out_specs=_row_spec,
    out_shape=jax.ShapeDtypeStruct((N, D), jnp.float32),
)


def kernel(x, edge_index, W1, b1, g1, be1, W2, b2, g2, be2, W3, b3, g3, be3):
    src = edge_index[0].astype(jnp.int32)
    dst = edge_index[1].astype(jnp.int32)
    e = src.shape[0]
    u = NS * EB
    nb0 = max(2, int(round(e * _R0 / u)))
    nb0 += nb0 % 2
    e0 = nb0 * u
    e1 = e - e0
    nb1 = max(2, -(-e1 // u))
    nb1 += nb1 % 2
    nbmax = -(-max(nb0, nb1) // 8) * 8

    def _core_arrays(idx, fill):
        a0 = idx[:e0].reshape(NS, nb0, EB)
        a0 = jnp.pad(a0, ((0, 0), (0, nbmax - nb0), (0, 0)),
                     constant_values=fill)
        a1 = jnp.concatenate([idx[e0:], jnp.full((nb1 * u - e1,), fill,
                                                 jnp.int32)])
        a1 = a1.reshape(NS, nb1, EB)
        a1 = jnp.pad(a1, ((0, 0), (0, nbmax - nb1), (0, 0)),
                     constant_values=fill)
        return jnp.stack([a0, a1])

    srcp = _core_arrays(src, 0)
    dstp = _core_arrays(dst, N)

    sc_deg, sc_scat = _make_sc_kernels(nbmax, nb0, nb1)

    degp = sc_deg(dstp)
    b1r, g1r, be1r = b1.reshape(1, D), g1.reshape(1, D), be1.reshape(1, D)
    b2r, g2r, be2r = b2.reshape(1, D), g2.reshape(1, D), be2.reshape(1, D)
    b3r, g3r, be3r = b3.reshape(1, D), g3.reshape(1, D), be3.reshape(1, D)

    hh1 = _premm(x, degp, W1)
    p1 = sc_scat(hh1, srcp, dstp)
    hh2 = _combmm(p1, hh1, degp, b1r, g1r, be1r, W2)
    p2 = sc_scat(hh2, srcp, dstp)
    hh3 = _combmm(p2, hh2, degp, b2r, g2r, be2r, W3)
    p3 = sc_scat(hh3, srcp, dstp)
    return _final(p3, hh3, degp, b3r, g3r, be3r)


# speedup vs baseline: 1.6927x; 1.3065x over previous
import functools

import jax
import jax.numpy as jnp
from jax import lax
from jax.experimental import pallas as pl
from jax.experimental.pallas import tpu as pltpu
from jax.experimental.pallas import tpu_sc as plsc

N = 10000
D = 128
EPS = 1e-3

NC = 2
NS = 16
EB = 128

N_PAD = 10112
ZROWS = N_PAD // NS
DEGW = 128
_R0 = 0.58

_mesh = plsc.VectorSubcoreMesh(core_axis_name="c", subcore_axis_name="s")


def _degree_body(nb0, nb1, dstp_hbm, out_hbm, dst_v, drow_v, ones_v, acc):
    c = lax.axis_index("c")
    s = lax.axis_index("s")
    nb = jnp.where(c == 0, nb0, nb1)
    pltpu.sync_copy(dstp_hbm.at[c, s], dst_v)

    e1 = jnp.full((16,), 1.0, jnp.float32)
    z16 = jnp.zeros((16,), jnp.float32)

    def zfill(i, _):
        for k in range(DEGW // 16):
            ones_v[i, pl.ds(k * 16, 16)] = z16
        return _

    lax.fori_loop(0, EB, zfill, None)
    for t in range(ZROWS // EB):
        pltpu.sync_copy(ones_v, acc.at[pl.ds(s * ZROWS + t * EB, EB)])
    rem = ZROWS % EB
    if rem:
        pltpu.sync_copy(ones_v.at[pl.ds(0, rem)],
                        acc.at[pl.ds(s * ZROWS + (ZROWS // EB) * EB, rem)])
    plsc.subcore_barrier()

    def fill(i, _):
        for k in range(DEGW // 16):
            ones_v[i, pl.ds(k * 16, 16)] = e1
        return _

    lax.fori_loop(0, EB, fill, None)

    def step(j, _):
        for k in range(EB // 16):
            drow_v[pl.ds(k * 16, 16)] = dst_v[j, pl.ds(k * 16, 16)]
        pltpu.sync_copy(ones_v, acc.at[drow_v], add=True)
        return _

    lax.fori_loop(0, nb, step, None)
    plsc.subcore_barrier()
    pltpu.sync_copy(acc.at[pl.ds(s * ZROWS, ZROWS)],
                    out_hbm.at[c, pl.ds(s * ZROWS, ZROWS)])


def _scatter_body(nb0, nb1, h_hbm, srcp_hbm, dstp_hbm, out_hbm,
                  srow_v, drow_v, sidx0_v, sidx1_v, pidx0_v, pidx1_v,
                  buf0, buf1, sem0, sem1, ssem0, ssem1, acc):
    c = lax.axis_index("c")
    s = lax.axis_index("s")
    nb = jnp.where(c == 0, nb0, nb1)

    z16 = jnp.zeros((16,), jnp.float32)

    def zfill(i, _):
        for k in range(D // 16):
            buf0[i, pl.ds(k * 16, 16)] = z16
        return _

    lax.fori_loop(0, EB, zfill, None)
    for t in range(ZROWS // EB):
        pltpu.sync_copy(buf0, acc.at[pl.ds(s * ZROWS + t * EB, EB)])
    rem = ZROWS % EB
    if rem:
        pltpu.sync_copy(buf0.at[pl.ds(0, rem)],
                        acc.at[pl.ds(s * ZROWS + (ZROWS // EB) * EB, rem)])
    plsc.subcore_barrier()

    def copy_row(dst_ref, src_ref):
        for k in range(EB // 16):
            dst_ref[pl.ds(k * 16, 16)] = src_ref[pl.ds(k * 16, 16)]

    npairs = nb // 2
    pltpu.sync_copy(srcp_hbm.at[c, s, 0], sidx0_v)
    pltpu.sync_copy(srcp_hbm.at[c, s, 1], sidx1_v)
    pltpu.sync_copy(dstp_hbm.at[c, s, 0], pidx0_v)
    pltpu.sync_copy(dstp_hbm.at[c, s, 1], pidx1_v)

    def pair(t, _):
        j0 = 2 * t
        j1 = j0 + 1

        @pl.when(t > 0)
        def _drain_prev():
            pltpu.make_async_copy(srcp_hbm.at[c, s, j0], sidx0_v, sem1).wait()
            pltpu.make_async_copy(srcp_hbm.at[c, s, j1], sidx1_v, sem1).wait()
            pltpu.make_async_copy(dstp_hbm.at[c, s, j0], pidx0_v, sem1).wait()
            pltpu.make_async_copy(dstp_hbm.at[c, s, j1], pidx1_v, sem1).wait()
            pltpu.make_async_copy(buf0, acc.at[srow_v], ssem0).wait()
            pltpu.make_async_copy(buf1, acc.at[drow_v], ssem1).wait()

        pltpu.async_copy(h_hbm.at[sidx0_v], buf0, sem0)
        pltpu.async_copy(h_hbm.at[sidx1_v], buf1, sem0)
        pltpu.make_async_copy(h_hbm.at[sidx0_v], buf0, sem0).wait()
        pltpu.make_async_copy(h_hbm.at[sidx1_v], buf1, sem0).wait()

        copy_row(srow_v, pidx0_v)
        copy_row(drow_v, pidx1_v)

        @pl.when(t + 1 < npairs)
        def _prefetch_idx():
            pltpu.async_copy(srcp_hbm.at[c, s, j0 + 2], sidx0_v, sem1)
            pltpu.async_copy(srcp_hbm.at[c, s, j1 + 2], sidx1_v, sem1)
            pltpu.async_copy(dstp_hbm.at[c, s, j0 + 2], pidx0_v, sem1)
            pltpu.async_copy(dstp_hbm.at[c, s, j1 + 2], pidx1_v, sem1)

        pltpu.async_copy(buf0, acc.at[srow_v], ssem0, add=True)
        pltpu.async_copy(buf1, acc.at[drow_v], ssem1, add=True)
        return _

    lax.fori_loop(0, npairs, pair, None)
    pltpu.make_async_copy(buf0, acc.at[srow_v], ssem0).wait()
    pltpu.make_async_copy(buf1, acc.at[drow_v], ssem1).wait()
    plsc.subcore_barrier()
    pltpu.sync_copy(acc.at[pl.ds(s * ZROWS, ZROWS)],
                    out_hbm.at[c, pl.ds(s * ZROWS, ZROWS)])


def _make_sc_kernels(nb, nb0, nb1):
    deg = pl.kernel(
        functools.partial(_degree_body, nb0, nb1),
        out_type=jax.ShapeDtypeStruct((NC, N_PAD, DEGW), jnp.float32),
        mesh=_mesh,
        scratch_types=[
            pltpu.VMEM((nb, EB), jnp.int32),
            pltpu.VMEM((EB,), jnp.int32),
            pltpu.VMEM((EB, DEGW), jnp.float32),
            pltpu.VMEM_SHARED((N_PAD, DEGW), jnp.float32),
        ],
    )
    scat = pl.kernel(
        functools.partial(_scatter_body, nb0, nb1),
        out_type=jax.ShapeDtypeStruct((NC, N_PAD, D), jnp.float32),
        mesh=_mesh,
        scratch_types=[
            pltpu.VMEM((EB,), jnp.int32),
            pltpu.VMEM((EB,), jnp.int32),
            pltpu.VMEM((EB,), jnp.int32),
            pltpu.VMEM((EB,), jnp.int32),
            pltpu.VMEM((EB,), jnp.int32),
            pltpu.VMEM((EB,), jnp.int32),
            pltpu.VMEM((EB, D), jnp.float32),
            pltpu.VMEM((EB, D), jnp.float32),
            pltpu.SemaphoreType.DMA,
            pltpu.SemaphoreType.DMA,
            pltpu.SemaphoreType.DMA,
            pltpu.SemaphoreType.DMA,
            pltpu.VMEM_SHARED((N_PAD, D), jnp.float32),
        ],
    )
    return deg, scat


_BN_C = float(1.0 / (1.0 + EPS) ** 0.5)
_RB = 1000


def _premm_body(x_ref, d_ref, w_ref, o_ref):
    sc = lax.rsqrt(1.0 + d_ref[0, :, 0:1] + d_ref[1, :, 0:1])
    o_ref[...] = jnp.dot(x_ref[...] * sc, w_ref[...],
                         preferred_element_type=jnp.float32)


def _combmm_body(p_ref, hh_ref, d_ref, b_ref, g_ref, be_ref, w_ref, o_ref):
    sc = lax.rsqrt(1.0 + d_ref[0, :, 0:1] + d_ref[1, :, 0:1])
    agg = sc * (p_ref[0] + p_ref[1] + hh_ref[...])
    y = (agg + b_ref[...]) * (g_ref[...] * _BN_C) + be_ref[...]
    h = jnp.maximum(y, 0.0)
    o_ref[...] = jnp.dot(h * sc, w_ref[...],
                         preferred_element_type=jnp.float32)


def _final_body(p_ref, hh_ref, d_ref, b_ref, g_ref, be_ref, o_ref):
    sc = lax.rsqrt(1.0 + d_ref[0, :, 0:1] + d_ref[1, :, 0:1])
    agg = sc * (p_ref[0] + p_ref[1] + hh_ref[...])
    y = (agg + b_ref[...]) * (g_ref[...] * _BN_C) + be_ref[...]
    o_ref[...] = jnp.maximum(y, 0.0)


_row_spec = pl.BlockSpec((_RB, D), lambda i: (i, 0))
_deg_spec = pl.BlockSpec((NC, _RB, DEGW), lambda i: (0, i, 0))
_part_spec = pl.BlockSpec((NC, _RB, D), lambda i: (0, i, 0))
_w_spec = pl.BlockSpec((D, D), lambda i: (0, 0))
_vec_spec = pl.BlockSpec((1, D), lambda i: (0, 0))

_GRID = (N // _RB,)

_premm = pl.pallas_call(
    _premm_body,
    grid=_GRID,
    in_specs=[_row_spec, _deg_spec, _w_spec],
    out_specs=_row_spec,
    out_shape=jax.ShapeDtypeStruct((N, D), jnp.float32),
)

_combmm = pl.pallas_call(
    _combmm_body,
    grid=_GRID,
    in_specs=[_part_spec, _row_spec, _deg_spec,
              _vec_spec, _vec_spec, _vec_spec, _w_spec],
    out_specs=_row_spec,
    out_shape=jax.ShapeDtypeStruct((N, D), jnp.float32),
)

_final = pl.pallas_call(
    _final_body,
    grid=_GRID,
    in_specs=[_part_spec, _row_spec, _deg_spec,
              _vec_spec, _vec_spec, _vec_spec],
    out_specs=_row_spec,
    out_shape=jax.ShapeDtypeStruct((N, D), jnp.float32),
)


def kernel(x, edge_index, W1, b1, g1, be1, W2, b2, g2, be2, W3, b3, g3, be3):
    src = edge_index[0].astype(jnp.int32)
    dst = edge_index[1].astype(jnp.int32)
    e = src.shape[0]
    u = NS * EB
    nb0 = max(2, int(round(e * _R0 / u)))
    nb0 += nb0 % 2
    e0 = nb0 * u
    e1 = e - e0
    nb1 = max(2, -(-e1 // u))
    nb1 += nb1 % 2
    nbmax = -(-max(nb0, nb1) // 8) * 8

    def _core_arrays(idx, fill):
        a0 = idx[:e0].reshape(NS, nb0, EB)
        a0 = jnp.pad(a0, ((0, 0), (0, nbmax - nb0), (0, 0)),
                     constant_values=fill)
        a1 = jnp.concatenate([idx[e0:], jnp.full((nb1 * u - e1,), fill,
                                                 jnp.int32)])
        a1 = a1.reshape(NS, nb1, EB)
        a1 = jnp.pad(a1, ((0, 0), (0, nbmax - nb1), (0, 0)),
                     constant_values=fill)
        return jnp.stack([a0, a1])

    srcp = _core_arrays(src, 0)
    dstp = _core_arrays(dst, N)

    sc_deg, sc_scat = _make_sc_kernels(nbmax, nb0, nb1)

    degp = sc_deg(dstp)
    b1r, g1r, be1r = b1.reshape(1, D), g1.reshape(1, D), be1.reshape(1, D)
    b2r, g2r, be2r = b2.reshape(1, D), g2.reshape(1, D), be2.reshape(1, D)
    b3r, g3r, be3r = b3.reshape(1, D), g3.reshape(1, D), be3.reshape(1, D)

    hh1 = _premm(x, degp, W1)
    p1 = sc_scat(hh1, srcp, dstp)
    hh2 = _combmm(p1, hh1, degp, b1r, g1r, be1r, W2)
    p2 = sc_scat(hh2, srcp, dstp)
    hh3 = _combmm(p2, hh2, degp, b2r, g2r, be2r, W3)
    p3 = sc_scat(hh3, srcp, dstp)
    return _final(p3, hh3, degp, b3r, g3r, be3r)
